# Initial kernel scaffold; baseline (speedup 1.0000x reference)
import jax
import jax.numpy as jnp
from jax.experimental import pallas as pl


def kernel(x_reactions, x_constraints, edge_index_rc, edge_index_cr, batch_mask, y_sign, Wir, bir, Wic, bic, sage_pW, sage_pb, sage_lW, sage_lb, sage_rW, W1, b1, g1, be1, W2, b2, g2, be2, W3, b3):
    raise NotImplementedError("write your pallas kernel here")


# speedup vs baseline: 4.1177x; 4.1177x over previous
import functools

import jax
import jax.numpy as jnp
from jax import lax
from jax.experimental import pallas as pl
from jax.experimental.pallas import tpu as pltpu
from jax.experimental.pallas import tpu_sc as plsc

N = 10000
F = 256
BLK = 1000
GRID = N // BLK
E_PAD = 163840
CH = 128
NCHUNK = E_PAD // (16 * CH)
ACC_ROWS = 10240
ROWS_PER_TILE = ACC_ROWS // 16
_F32 = jnp.float32


def _sc_agg_body(h4, src4, dst3, z64, z16, ones16,
                 agg, cnt,
                 sidx, didx, rows0, rows1, zbuf, z16buf, obuf, acc, cacc,
                 sem0, sem1):
    c = lax.axis_index("c")
    s = lax.axis_index("s")
    pltpu.sync_copy(dst3.at[s], didx)
    pltpu.sync_copy(z64, zbuf)
    pltpu.sync_copy(z16, z16buf)
    pltpu.sync_copy(ones16, obuf)
    base = s * ROWS_PER_TILE
    bufs = ((rows0, sem0), (rows1, sem1))

    for q in range(2):
        p = 2 * c + q
        pltpu.sync_copy(src4.at[p, s], sidx)
        for k in range(ROWS_PER_TILE // CH):
            pltpu.sync_copy(zbuf, acc.at[pl.ds(base + k * CH, CH)])

        if q == 0:
            @pl.when(c == 0)
            def _():
                for k in range(ROWS_PER_TILE // CH):
                    pltpu.sync_copy(z16buf, cacc.at[pl.ds(base + k * CH, CH)])

        plsc.subcore_barrier()

        def body(g, carry):
            for b in range(2):
                j = 2 * g + b
                rows, sem = bufs[b]
                pltpu.async_copy(h4.at[sidx.at[j]], rows, sem)
            for b in range(2):
                j = 2 * g + b
                rows, sem = bufs[b]
                pltpu.make_async_copy(h4.at[sidx.at[j]], rows, sem).wait()
                pltpu.sync_copy(rows, acc.at[didx.at[j]], add=True)
                if q == 0:
                    @pl.when(c == 0)
                    def _():
                        pltpu.sync_copy(obuf, cacc.at[didx.at[j]], add=True)
            return carry

        lax.fori_loop(0, NCHUNK // 2, body, 0)
        plsc.subcore_barrier()
        for k in range(ROWS_PER_TILE // CH):
            sl = pl.ds(base + k * CH, CH)
            pltpu.sync_copy(acc.at[sl], agg.at[p, sl])

        if q == 0:
            @pl.when(c == 0)
            def _():
                for k in range(ROWS_PER_TILE // CH):
                    sl = pl.ds(base + k * CH, CH)
                    pltpu.sync_copy(cacc.at[sl], cnt.at[sl])


@functools.cache
def _get_sc_agg():
    return pl.kernel(
        _sc_agg_body,
        out_type=(
            jax.ShapeDtypeStruct((4, ACC_ROWS, 64), _F32),
            jax.ShapeDtypeStruct((ACC_ROWS, 16), _F32),
        ),
        mesh=plsc.VectorSubcoreMesh(core_axis_name="c", subcore_axis_name="s"),
        compiler_params=pltpu.CompilerParams(use_tc_tiling_on_sc=False),
        scratch_types=[
        pltpu.VMEM((NCHUNK, CH), jnp.int32),
        pltpu.VMEM((NCHUNK, CH), jnp.int32),
        pltpu.VMEM((CH, 64), _F32),
        pltpu.VMEM((CH, 64), _F32),
        pltpu.VMEM((CH, 64), _F32),
        pltpu.VMEM((CH, 16), _F32),
        pltpu.VMEM((CH, 16), _F32),
        pltpu.VMEM_SHARED((ACC_ROWS, 64), _F32),
        pltpu.VMEM_SHARED((ACC_ROWS, 16), _F32),
            pltpu.SemaphoreType.DMA,
            pltpu.SemaphoreType.DMA,
        ],
    )


def _bdot(a, b):
    return jnp.dot(a.astype(jnp.bfloat16), b.astype(jnp.bfloat16),
                   preferred_element_type=_F32)


def _lin_kernel(x_ref, wt_ref, b_ref, o_ref, *, act):
    y = _bdot(x_ref[...], wt_ref[...])
    y = y + b_ref[...]
    if act:
        y = jnp.maximum(y, 0.0)
    o_ref[...] = y


def _linear(x, wt, b, act=False):
    m, k = x.shape
    n = wt.shape[1]
    return pl.pallas_call(
        functools.partial(_lin_kernel, act=act),
        grid=(m // BLK,),
        in_specs=[
            pl.BlockSpec((BLK, k), lambda i: (i, 0)),
            pl.BlockSpec((k, n), lambda i: (0, 0)),
            pl.BlockSpec((1, n), lambda i: (0, 0)),
        ],
        out_specs=pl.BlockSpec((BLK, n), lambda i: (i, 0)),
        out_shape=jax.ShapeDtypeStruct((m, n), _F32),
    )(x, wt, b)


def _sage_out_kernel(a0_ref, a1_ref, a2_ref, a3_ref, cnt_ref, xd_ref,
                     lwt_ref, lb_ref, rwt_ref, o_ref):
    cn = jnp.maximum(cnt_ref[...][:, 0:1], 1.0)
    s = jnp.concatenate([a0_ref[0], a1_ref[0], a2_ref[0], a3_ref[0]], axis=1)
    a = s / cn
    out = _bdot(a, lwt_ref[...]) + lb_ref[...]
    out = out + _bdot(xd_ref[...], rwt_ref[...])
    nrm = jnp.sqrt(jnp.sum(out * out, axis=1, keepdims=True))
    o_ref[...] = out / jnp.maximum(nrm, 1e-12)


def _sage_out(agg, cnt, x_dst, lwt, lb, rwt):
    return pl.pallas_call(
        _sage_out_kernel,
        grid=(GRID,),
        in_specs=[
            pl.BlockSpec((1, BLK, 64), lambda i: (0, i, 0)),
            pl.BlockSpec((1, BLK, 64), lambda i: (1, i, 0)),
            pl.BlockSpec((1, BLK, 64), lambda i: (2, i, 0)),
            pl.BlockSpec((1, BLK, 64), lambda i: (3, i, 0)),
            pl.BlockSpec((BLK, 16), lambda i: (i, 0)),
            pl.BlockSpec((BLK, F), lambda i: (i, 0)),
            pl.BlockSpec((F, F), lambda i: (0, 0)),
            pl.BlockSpec((1, F), lambda i: (0, 0)),
            pl.BlockSpec((F, F), lambda i: (0, 0)),
        ],
        out_specs=pl.BlockSpec((BLK, F), lambda i: (i, 0)),
        out_shape=jax.ShapeDtypeStruct((N, F), _F32),
    )(agg, agg, agg, agg, cnt, x_dst, lwt, lb, rwt)


def _mlp1_kernel(x_ref, wt_ref, b_ref, y_ref, s_ref):
    i = pl.program_id(0)
    y = _bdot(x_ref[...], wt_ref[...])
    y = y + b_ref[...]
    y_ref[...] = y

    @pl.when(i == 0)
    def _():
        s_ref[...] = jnp.zeros_like(s_ref)

    s_ref[...] += jnp.sum(y, axis=0, keepdims=True)


def _mlp1(x, wt, b):
    return pl.pallas_call(
        _mlp1_kernel,
        grid=(GRID,),
        in_specs=[
            pl.BlockSpec((BLK, F), lambda i: (i, 0)),
            pl.BlockSpec((F, F), lambda i: (0, 0)),
            pl.BlockSpec((1, F), lambda i: (0, 0)),
        ],
        out_specs=[
            pl.BlockSpec((BLK, F), lambda i: (i, 0)),
            pl.BlockSpec((1, F), lambda i: (0, 0)),
        ],
        out_shape=[
            jax.ShapeDtypeStruct((N, F), _F32),
            jax.ShapeDtypeStruct((1, F), _F32),
        ],
    )(x, wt, b)


def _bnstat_kernel(y_ref, s_ref, v_ref):
    i = pl.program_id(0)

    @pl.when(i == 0)
    def _():
        v_ref[...] = jnp.zeros_like(v_ref)

    d = y_ref[...] - s_ref[...] * (1.0 / N)
    v_ref[...] += jnp.sum(d * d, axis=0, keepdims=True)


def _bnstat(y, s):
    return pl.pallas_call(
        _bnstat_kernel,
        grid=(GRID,),
        in_specs=[
            pl.BlockSpec((BLK, F), lambda i: (i, 0)),
            pl.BlockSpec((1, F), lambda i: (0, 0)),
        ],
        out_specs=pl.BlockSpec((1, F), lambda i: (0, 0)),
        out_shape=jax.ShapeDtypeStruct((1, F), _F32),
    )(y, s)


def _bn(y, s, vs, g, be):
    m = s * (1.0 / N)
    v = vs * (1.0 / N)
    return (y - m) / jnp.sqrt(v + 1e-5) * g + be


def _mlp2_kernel(y1_ref, s1_ref, vs1_ref, g_ref, be_ref, wt_ref, b_ref,
                 y_ref, s_ref):
    i = pl.program_id(0)
    z = jnp.maximum(_bn(y1_ref[...], s1_ref[...], vs1_ref[...], g_ref[...],
                        be_ref[...]), 0.0)
    y = _bdot(z, wt_ref[...]) + b_ref[...]
    y_ref[...] = y

    @pl.when(i == 0)
    def _():
        s_ref[...] = jnp.zeros_like(s_ref)

    s_ref[...] += jnp.sum(y, axis=0, keepdims=True)


def _mlp2(y1, s1, vs1, g, be, wt, b):
    vec = pl.BlockSpec((1, F), lambda i: (0, 0))
    return pl.pallas_call(
        _mlp2_kernel,
        grid=(GRID,),
        in_specs=[
            pl.BlockSpec((BLK, F), lambda i: (i, 0)),
            vec, vec, vec, vec,
            pl.BlockSpec((F, F), lambda i: (0, 0)),
            vec,
        ],
        out_specs=[
            pl.BlockSpec((BLK, F), lambda i: (i, 0)),
            pl.BlockSpec((1, F), lambda i: (0, 0)),
        ],
        out_shape=[
            jax.ShapeDtypeStruct((N, F), _F32),
            jax.ShapeDtypeStruct((1, F), _F32),
        ],
    )(y1, s1, vs1, g, be, wt, b)


def _pool_kernel(y2_ref, s2_ref, vs2_ref, g_ref, be_ref, bm_ref, w3t_ref,
                 b3_ref, ys_ref, ps_ref, pc_ref, res_ref):
    i = pl.program_id(0)
    z = jnp.maximum(_bn(y2_ref[...], s2_ref[...], vs2_ref[...], g_ref[...],
                        be_ref[...]), 0.0)
    gids = jax.lax.broadcasted_iota(jnp.int32, (1, 64), 1)
    oh = (bm_ref[...] == gids).astype(_F32)

    @pl.when(i == 0)
    def _():
        ps_ref[...] = jnp.zeros_like(ps_ref)
        pc_ref[...] = jnp.zeros_like(pc_ref)

    ps_ref[...] += lax.dot_general(oh, z, (((0,), (0,)), ((), ())),
                                   preferred_element_type=_F32,
                                   precision=jax.lax.Precision.HIGHEST)
    pc_ref[...] += lax.dot_general(oh, jnp.ones((BLK, 1), _F32),
                                   (((0,), (0,)), ((), ())),
                                   preferred_element_type=_F32, precision=jax.lax.Precision.HIGHEST)

    @pl.when(i == GRID - 1)
    def _():
        pooled = ps_ref[...] / jnp.maximum(pc_ref[...], 1.0)
        r = _bdot(pooled, w3t_ref[...])
        res_ref[...] = (r + b3_ref[...]) * ys_ref[...]


def _pool(y2, s2, vs2, g, be, bm, w3t, b3, ys):
    vec = pl.BlockSpec((1, F), lambda i: (0, 0))
    return pl.pallas_call(
        _pool_kernel,
        grid=(GRID,),
        in_specs=[
            pl.BlockSpec((BLK, F), lambda i: (i, 0)),
            vec, vec, vec, vec,
            pl.BlockSpec((BLK, 1), lambda i: (i, 0)),
            pl.BlockSpec((F, 1), lambda i: (0, 0)),
            pl.BlockSpec((1, 1), lambda i: (0, 0)),
            pl.BlockSpec((64, 1), lambda i: (0, 0)),
        ],
        out_specs=[
            pl.BlockSpec((64, F), lambda i: (0, 0)),
            pl.BlockSpec((64, 1), lambda i: (0, 0)),
            pl.BlockSpec((64, 1), lambda i: (0, 0)),
        ],
        out_shape=[
            jax.ShapeDtypeStruct((64, F), _F32),
            jax.ShapeDtypeStruct((64, 1), _F32),
            jax.ShapeDtypeStruct((64, 1), _F32),
        ],
    )(y2, s2, vs2, g, be, bm, w3t, b3, ys)


def _prep_edges(ei):
    src, dst = ei[0].astype(jnp.int32), ei[1].astype(jnp.int32)
    e = src.shape[0]
    npad = E_PAD - e
    pad_src = (jnp.arange(npad, dtype=jnp.int32) * 37) % N
    pad_dst = N + (jnp.arange(npad, dtype=jnp.int32) % (ACC_ROWS - N))
    src = jnp.concatenate([src, pad_src])
    dst = jnp.concatenate([dst, pad_dst])
    s4 = 4 * src
    src4 = jnp.stack([s4, s4 + 1, s4 + 2, s4 + 3]).reshape(4, 16, NCHUNK, CH)
    dst3 = dst.reshape(16, NCHUNK, CH)
    return src4, dst3


def _sage_layer(x_src, x_dst, src4, dst3, consts, pwt, pb, lwt, lb, rwt):
    h = _linear(x_src, pwt, pb, act=True)
    h4 = h.reshape(4 * N, 64)
    agg, cnt = _get_sc_agg()(h4, src4, dst3, *consts)
    return _sage_out(agg, cnt, x_dst, lwt, lb, rwt)


def kernel(x_reactions, x_constraints, edge_index_rc, edge_index_cr,
           batch_mask, y_sign, Wir, bir, Wic, bic, sage_pW, sage_pb,
           sage_lW, sage_lb, sage_rW, W1, b1, g1, be1, W2, b2, g2, be2,
           W3, b3):
    src2_rc, dst3_rc = _prep_edges(edge_index_rc)
    src2_cr, dst3_cr = _prep_edges(edge_index_cr)
    consts = (jnp.zeros((CH, 64), _F32), jnp.zeros((CH, 16), _F32),
              jnp.ones((CH, 16), _F32))

    xr = _linear(x_reactions, Wir.T, bir[None, :])
    xc = _linear(x_constraints, Wic.T, bic[None, :])

    edges = ((src2_rc, dst3_rc), (src2_cr, dst3_cr))
    xs = [xr, xc]
    for k in range(4):
        par = k % 2
        src2, dst3 = edges[par]
        x_src, x_dst = xs[par], xs[1 - par]
        xs[1 - par] = _sage_layer(
            x_src, x_dst, src2, dst3, consts,
            sage_pW[k].T, sage_pb[k][None, :],
            sage_lW[k].T, sage_lb[k][None, :], sage_rW[k].T)

    xr = xs[0]
    y1, s1 = _mlp1(xr, W1.T, b1[None, :])
    vs1 = _bnstat(y1, s1)
    y2, s2 = _mlp2(y1, s1, vs1, g1[None, :], be1[None, :],
                   W2.T, b2[None, :])
    vs2 = _bnstat(y2, s2)
    _, _, res = _pool(y2, s2, vs2, g2[None, :], be2[None, :],
                      batch_mask.astype(jnp.int32)[:, None],
                      W3.T, b3[None, :], y_sign[:, None])
    return res[:, 0]
